# baseline (speedup 1.0000x reference)
import jax
import jax.numpy as jnp
from jax.experimental import pallas as pl
from jax.experimental.pallas import tpu as pltpu

_F32 = jnp.float32


def _embed_body(x0, x1, x2, w0, b0, w1, b1, w2, b2, h0, h1, h2):
    h0[...] = jnp.dot(x0[...], w0[...], preferred_element_type=_F32) + b0[...]
    h1[...] = jnp.dot(x1[...], w1[...], preferred_element_type=_F32) + b1[...]
    h2[...] = jnp.dot(x2[...], w2[...], preferred_element_type=_F32) + b2[...]


def _embed(x0, x1, x2, W0, b0, W1, b1, W2, b2):
    C = W0.shape[1]
    outs = [jax.ShapeDtypeStruct((x.shape[0], C), _F32) for x in (x0, x1, x2)]
    return pl.pallas_call(_embed_body, out_shape=outs)(
        x0, x1, x2, W0, b0.reshape(1, -1), W1, b1.reshape(1, -1), W2, b2.reshape(1, -1)
    )


def _inc_body(B_ref, xs_ref, xd_ref, f_ref, bwd_ref):
    i = pl.program_id(0)
    blk = B_ref[...]
    f_ref[...] = jnp.dot(blk, xs_ref[...], preferred_element_type=_F32)

    @pl.when(i == 0)
    def _():
        bwd_ref[...] = jnp.zeros_like(bwd_ref)

    bwd_ref[...] += jax.lax.dot_general(
        blk, xd_ref[...], dimension_numbers=(((0,), (0,)), ((), ())),
        preferred_element_type=_F32)


def _incidence(B, xs, xd, R=256):
    Nr, Nc = B.shape
    C = xs.shape[1]
    return pl.pallas_call(
        _inc_body,
        grid=(Nr // R,),
        in_specs=[
            pl.BlockSpec((R, Nc), lambda i: (i, 0)),
            pl.BlockSpec((Nc, C), lambda i: (0, 0)),
            pl.BlockSpec((R, C), lambda i: (i, 0)),
        ],
        out_specs=[
            pl.BlockSpec((R, C), lambda i: (i, 0)),
            pl.BlockSpec((Nc, C), lambda i: (0, 0)),
        ],
        out_shape=[
            jax.ShapeDtypeStruct((Nr, C), _F32),
            jax.ShapeDtypeStruct((Nc, C), _F32),
        ],
        compiler_params=pltpu.CompilerParams(dimension_semantics=("arbitrary",)),
    )(B, xs, xd)


def _cheby_fused(L, srcs, wt, include_id, init=None, R=256):
    n_src = len(srcs)
    N = L.shape[0]
    C = srcs[0].shape[1]
    W = C * n_src
    K, _, C_OUT = wt.shape
    m = K // n_src - (1 if include_id else 0)
    nR = N // R
    has_init = init is not None

    def body(*refs):
        L_ref = refs[0]
        src_refs = refs[1:1 + n_src]
        wt_ref = refs[1 + n_src]
        init_ref = refs[2 + n_src] if has_init else None
        y_ref = refs[2 + n_src + int(has_init)]
        chain = refs[3 + n_src + int(has_init)]
        p = pl.program_id(0)
        i = pl.program_id(1)

        @pl.when((p == 0) & (i == 0))
        def _():
            for s in range(n_src):
                chain[0, :, s * C:(s + 1) * C] = src_refs[s][...]

        rows = pl.ds(i * R, R)
        new = jnp.dot(L_ref[rows, :], chain[p], preferred_element_type=_F32)
        chain[p + 1, rows, :] = new

        @pl.when(p == m - 1)
        def _():
            acc = init_ref[rows, :] if has_init else jnp.zeros((R, C_OUT), _F32)
            k = 0
            for s in range(n_src):
                cs = slice(s * C, (s + 1) * C)
                if include_id:
                    acc += jnp.dot(chain[0, rows, cs], wt_ref[k],
                                   preferred_element_type=_F32)
                    k += 1
                for j in range(1, m + 1):
                    t = new[:, cs] if j == m else chain[j, rows, cs]
                    acc += jnp.dot(t, wt_ref[k], preferred_element_type=_F32)
                    k += 1
            y_ref[rows, :] = acc

    in_specs = (
        [pl.BlockSpec((N, N), lambda p, i: (0, 0))]
        + [pl.BlockSpec((N, C), lambda p, i: (0, 0)) for _ in srcs]
        + [pl.BlockSpec(wt.shape, lambda p, i: (0, 0, 0))]
    )
    operands = [L] + list(srcs) + [wt]
    if has_init:
        in_specs.append(pl.BlockSpec((N, C_OUT), lambda p, i: (0, 0)))
        operands.append(init)
    return pl.pallas_call(
        body,
        grid=(m, nR),
        in_specs=in_specs,
        out_specs=pl.BlockSpec((N, C_OUT), lambda p, i: (0, 0)),
        out_shape=jax.ShapeDtypeStruct((N, C_OUT), _F32),
        scratch_shapes=[pltpu.VMEM((m + 1, N, W), _F32)],
        compiler_params=pltpu.CompilerParams(
            dimension_semantics=("arbitrary", "arbitrary"),
            vmem_limit_bytes=100 * 1024 * 1024),
    )(*operands)


def _logits_body(h_ref, w_ref, b_ref, o_ref):
    o_ref[...] = jax.nn.sigmoid(
        jnp.dot(h_ref[...], w_ref[...], preferred_element_type=_F32) + b_ref[...])


def _logits(h, W, b):
    return pl.pallas_call(
        _logits_body,
        out_shape=jax.ShapeDtypeStruct((h.shape[0], W.shape[1]), _F32),
    )(h, W, b.reshape(1, -1))


def kernel(x_0, x_1, x_2, laplacian_0, laplacian_down_1, laplacian_up_1,
           laplacian_2, incidence_1, incidence_2, in_W0, in_b0, in_W1, in_b1,
           in_W2, in_b2, w0_l0, w1_l0, w2_l0, w0_l1, w1_l1, w2_l1,
           out_W, out_b):
    h0, h1, h2 = _embed(x_0, x_1, x_2, in_W0, in_b0, in_W1, in_b1, in_W2, in_b2)
    idx_d = jnp.array([0, 1, 2, 5, 6, 7, 10, 11, 12], jnp.int32)
    idx_u = jnp.array([3, 4, 8, 9, 13, 14], jnp.int32)
    for (w0, w1, w2) in ((w0_l0, w1_l0, w2_l0), (w0_l1, w1_l1, w2_l1)):
        t01, t10 = _incidence(incidence_1, h1, h0)
        t12, t21 = _incidence(incidence_2, h2, h1)
        w0t = jnp.transpose(w0, (2, 0, 1))
        w1t = jnp.transpose(w1, (2, 0, 1))
        w2t = jnp.transpose(w2, (2, 0, 1))
        y0 = _cheby_fused(laplacian_0, [h0, t01], w0t, include_id=True)
        y1d = _cheby_fused(laplacian_down_1, [h1, t10, t12], w1t[idx_d],
                           include_id=True)
        y1 = _cheby_fused(laplacian_up_1, [h1, t10, t12], w1t[idx_u],
                          include_id=False, init=y1d)
        y2 = _cheby_fused(laplacian_2, [h2, t21], w2t, include_id=True)
        h0, h1, h2 = y0, y1, y2
    return _logits(h0, out_W, out_b)


# speedup vs baseline: 1.0639x; 1.0639x over previous
import jax
import jax.numpy as jnp
from jax.experimental import pallas as pl
from jax.experimental.pallas import tpu as pltpu

_F32 = jnp.float32
_BF16 = jnp.bfloat16


def _b(v):
    return v.astype(_BF16)


def _embed_body(x0, x1, x2, w0, b0, w1, b1, w2, b2, h0, h1, h2):
    h0[...] = jnp.dot(_b(x0[...]), _b(w0[...]), preferred_element_type=_F32) + b0[...]
    h1[...] = jnp.dot(_b(x1[...]), _b(w1[...]), preferred_element_type=_F32) + b1[...]
    h2[...] = jnp.dot(_b(x2[...]), _b(w2[...]), preferred_element_type=_F32) + b2[...]


def _embed(x0, x1, x2, W0, b0, W1, b1, W2, b2):
    C = W0.shape[1]
    outs = [jax.ShapeDtypeStruct((x.shape[0], C), _F32) for x in (x0, x1, x2)]
    return pl.pallas_call(_embed_body, out_shape=outs)(
        x0, x1, x2, W0, b0.reshape(1, -1), W1, b1.reshape(1, -1), W2, b2.reshape(1, -1)
    )


def _inc_body(B_ref, xs_ref, xd_ref, f_ref, bwd_ref):
    i = pl.program_id(0)
    blk = _b(B_ref[...])
    f_ref[...] = jnp.dot(blk, _b(xs_ref[...]), preferred_element_type=_F32)

    @pl.when(i == 0)
    def _():
        bwd_ref[...] = jnp.zeros_like(bwd_ref)

    bwd_ref[...] += jax.lax.dot_general(
        blk, _b(xd_ref[...]), dimension_numbers=(((0,), (0,)), ((), ())),
        preferred_element_type=_F32)


def _incidence(B, xs, xd, R=256):
    Nr, Nc = B.shape
    C = xs.shape[1]
    return pl.pallas_call(
        _inc_body,
        grid=(Nr // R,),
        in_specs=[
            pl.BlockSpec((R, Nc), lambda i: (i, 0)),
            pl.BlockSpec((Nc, C), lambda i: (0, 0)),
            pl.BlockSpec((R, C), lambda i: (i, 0)),
        ],
        out_specs=[
            pl.BlockSpec((R, C), lambda i: (i, 0)),
            pl.BlockSpec((Nc, C), lambda i: (0, 0)),
        ],
        out_shape=[
            jax.ShapeDtypeStruct((Nr, C), _F32),
            jax.ShapeDtypeStruct((Nc, C), _F32),
        ],
        compiler_params=pltpu.CompilerParams(dimension_semantics=("arbitrary",)),
    )(B, xs, xd)


def _cheby_fused(L, srcs, wt, include_id, init=None, R=256):
    n_src = len(srcs)
    N = L.shape[0]
    C = srcs[0].shape[1]
    W = C * n_src
    K, _, C_OUT = wt.shape
    m = K // n_src - (1 if include_id else 0)
    nR = N // R
    has_init = init is not None

    def body(*refs):
        L_ref = refs[0]
        src_refs = refs[1:1 + n_src]
        wt_ref = refs[1 + n_src]
        init_ref = refs[2 + n_src] if has_init else None
        y_ref = refs[2 + n_src + int(has_init)]
        chain = refs[3 + n_src + int(has_init)]
        p = pl.program_id(0)
        i = pl.program_id(1)

        @pl.when((p == 0) & (i == 0))
        def _():
            for s in range(n_src):
                chain[0, :, s * C:(s + 1) * C] = src_refs[s][...]

        rows = pl.ds(i * R, R)
        new = jnp.dot(_b(L_ref[rows, :]), _b(chain[p]), preferred_element_type=_F32)
        chain[p + 1, rows, :] = new

        @pl.when(p == m - 1)
        def _():
            acc = init_ref[rows, :] if has_init else jnp.zeros((R, C_OUT), _F32)
            k = 0
            for s in range(n_src):
                cs = slice(s * C, (s + 1) * C)
                if include_id:
                    acc += jnp.dot(_b(chain[0, rows, cs]), _b(wt_ref[k]),
                                   preferred_element_type=_F32)
                    k += 1
                for j in range(1, m + 1):
                    t = new[:, cs] if j == m else chain[j, rows, cs]
                    acc += jnp.dot(_b(t), _b(wt_ref[k]), preferred_element_type=_F32)
                    k += 1
            y_ref[rows, :] = acc

    in_specs = (
        [pl.BlockSpec((N, N), lambda p, i: (0, 0))]
        + [pl.BlockSpec((N, C), lambda p, i: (0, 0)) for _ in srcs]
        + [pl.BlockSpec(wt.shape, lambda p, i: (0, 0, 0))]
    )
    operands = [L] + list(srcs) + [wt]
    if has_init:
        in_specs.append(pl.BlockSpec((N, C_OUT), lambda p, i: (0, 0)))
        operands.append(init)
    return pl.pallas_call(
        body,
        grid=(m, nR),
        in_specs=in_specs,
        out_specs=pl.BlockSpec((N, C_OUT), lambda p, i: (0, 0)),
        out_shape=jax.ShapeDtypeStruct((N, C_OUT), _F32),
        scratch_shapes=[pltpu.VMEM((m + 1, N, W), _F32)],
        compiler_params=pltpu.CompilerParams(
            dimension_semantics=("arbitrary", "arbitrary"),
            vmem_limit_bytes=100 * 1024 * 1024),
    )(*operands)


def _logits_body(h_ref, w_ref, b_ref, o_ref):
    o_ref[...] = jax.nn.sigmoid(
        jnp.dot(_b(h_ref[...]), _b(w_ref[...]), preferred_element_type=_F32) + b_ref[...])


def _logits(h, W, b):
    return pl.pallas_call(
        _logits_body,
        out_shape=jax.ShapeDtypeStruct((h.shape[0], W.shape[1]), _F32),
    )(h, W, b.reshape(1, -1))


def kernel(x_0, x_1, x_2, laplacian_0, laplacian_down_1, laplacian_up_1,
           laplacian_2, incidence_1, incidence_2, in_W0, in_b0, in_W1, in_b1,
           in_W2, in_b2, w0_l0, w1_l0, w2_l0, w0_l1, w1_l1, w2_l1,
           out_W, out_b):
    h0, h1, h2 = _embed(x_0, x_1, x_2, in_W0, in_b0, in_W1, in_b1, in_W2, in_b2)
    idx_d = jnp.array([0, 1, 2, 5, 6, 7, 10, 11, 12], jnp.int32)
    idx_u = jnp.array([3, 4, 8, 9, 13, 14], jnp.int32)
    for (w0, w1, w2) in ((w0_l0, w1_l0, w2_l0), (w0_l1, w1_l1, w2_l1)):
        t01, t10 = _incidence(incidence_1, h1, h0)
        t12, t21 = _incidence(incidence_2, h2, h1)
        w0t = jnp.transpose(w0, (2, 0, 1))
        w1t = jnp.transpose(w1, (2, 0, 1))
        w2t = jnp.transpose(w2, (2, 0, 1))
        y0 = _cheby_fused(laplacian_0, [h0, t01], w0t, include_id=True)
        y1d = _cheby_fused(laplacian_down_1, [h1, t10, t12], w1t[idx_d],
                           include_id=True)
        y1 = _cheby_fused(laplacian_up_1, [h1, t10, t12], w1t[idx_u],
                          include_id=False, init=y1d)
        y2 = _cheby_fused(laplacian_2, [h2, t21], w2t, include_id=True)
        h0, h1, h2 = y0, y1, y2
    return _logits(h0, out_W, out_b)


# speedup vs baseline: 1.0722x; 1.0078x over previous
import jax
import jax.numpy as jnp
from jax.experimental import pallas as pl
from jax.experimental.pallas import tpu as pltpu

_F32 = jnp.float32
_BF16 = jnp.bfloat16


def _b(v):
    return v.astype(_BF16)


def _embed_body(x0, x1, x2, w0, b0, w1, b1, w2, b2, h0, h1, h2):
    h0[...] = jnp.dot(_b(x0[...]), _b(w0[...]), preferred_element_type=_F32) + b0[...]
    h1[...] = jnp.dot(_b(x1[...]), _b(w1[...]), preferred_element_type=_F32) + b1[...]
    h2[...] = jnp.dot(_b(x2[...]), _b(w2[...]), preferred_element_type=_F32) + b2[...]


def _embed(x0, x1, x2, W0, b0, W1, b1, W2, b2):
    C = W0.shape[1]
    outs = [jax.ShapeDtypeStruct((x.shape[0], C), _F32) for x in (x0, x1, x2)]
    return pl.pallas_call(_embed_body, out_shape=outs)(
        x0, x1, x2, W0, b0.reshape(1, -1), W1, b1.reshape(1, -1), W2, b2.reshape(1, -1)
    )


def _inc_body(B_ref, xs_ref, xd_ref, f_ref, bwd_ref):
    i = pl.program_id(0)
    blk = _b(B_ref[...])
    f_ref[...] = jnp.dot(blk, _b(xs_ref[...]), preferred_element_type=_F32)

    @pl.when(i == 0)
    def _():
        bwd_ref[...] = jnp.zeros_like(bwd_ref)

    bwd_ref[...] += jax.lax.dot_general(
        blk, _b(xd_ref[...]), dimension_numbers=(((0,), (0,)), ((), ())),
        preferred_element_type=_F32)


def _incidence(B, xs, xd, R=256):
    Nr, Nc = B.shape
    C = xs.shape[1]
    return pl.pallas_call(
        _inc_body,
        grid=(Nr // R,),
        in_specs=[
            pl.BlockSpec((R, Nc), lambda i: (i, 0)),
            pl.BlockSpec((Nc, C), lambda i: (0, 0)),
            pl.BlockSpec((R, C), lambda i: (i, 0)),
        ],
        out_specs=[
            pl.BlockSpec((R, C), lambda i: (i, 0)),
            pl.BlockSpec((Nc, C), lambda i: (0, 0)),
        ],
        out_shape=[
            jax.ShapeDtypeStruct((Nr, C), _F32),
            jax.ShapeDtypeStruct((Nc, C), _F32),
        ],
        compiler_params=pltpu.CompilerParams(dimension_semantics=("arbitrary",)),
    )(B, xs, xd)


def _cheby_fused(L, srcs, wt, include_id, init=None, R=256):
    n_src = len(srcs)
    N = L.shape[0]
    C = srcs[0].shape[1]
    W = C * n_src
    K, _, C_OUT = wt.shape
    m = K // n_src - (1 if include_id else 0)
    nR = N // R
    has_init = init is not None

    def body(*refs):
        L_ref = refs[0]
        src_refs = refs[1:1 + n_src]
        wt_ref = refs[1 + n_src]
        init_ref = refs[2 + n_src] if has_init else None
        y_ref = refs[2 + n_src + int(has_init)]
        Lb = refs[3 + n_src + int(has_init)]
        chain = refs[4 + n_src + int(has_init)]
        p = pl.program_id(0)
        i = pl.program_id(1)

        @pl.when((p == 0) & (i == 0))
        def _():
            for s in range(n_src):
                chain[0, :, s * C:(s + 1) * C] = src_refs[s][...]

        rows = pl.ds(i * R, R)

        @pl.when(p == 0)
        def _():
            blk = _b(L_ref[...])
            Lb[rows, :] = blk
            chain[1, rows, :] = jnp.dot(blk, _b(chain[0]),
                                        preferred_element_type=_F32)

        @pl.when(p > 0)
        def _():
            chain[p + 1, rows, :] = jnp.dot(Lb[rows, :], _b(chain[p]),
                                            preferred_element_type=_F32)

        @pl.when(p == m - 1)
        def _():
            acc = init_ref[rows, :] if has_init else jnp.zeros((R, C_OUT), _F32)
            k = 0
            for s in range(n_src):
                cs = slice(s * C, (s + 1) * C)
                if include_id:
                    acc += jnp.dot(_b(chain[0, rows, cs]), _b(wt_ref[k]),
                                   preferred_element_type=_F32)
                    k += 1
                for j in range(1, m + 1):
                    acc += jnp.dot(_b(chain[j, rows, cs]), _b(wt_ref[k]),
                                   preferred_element_type=_F32)
                    k += 1
            y_ref[rows, :] = acc

    in_specs = (
        [pl.BlockSpec((R, N), lambda p, i: (jnp.where(p == 0, i, 0), 0))]
        + [pl.BlockSpec((N, C), lambda p, i: (0, 0)) for _ in srcs]
        + [pl.BlockSpec(wt.shape, lambda p, i: (0, 0, 0))]
    )
    operands = [L] + list(srcs) + [wt]
    if has_init:
        in_specs.append(pl.BlockSpec((N, C_OUT), lambda p, i: (0, 0)))
        operands.append(init)
    return pl.pallas_call(
        body,
        grid=(m, nR),
        in_specs=in_specs,
        out_specs=pl.BlockSpec((N, C_OUT), lambda p, i: (0, 0)),
        out_shape=jax.ShapeDtypeStruct((N, C_OUT), _F32),
        scratch_shapes=[pltpu.VMEM((N, N), _BF16),
                        pltpu.VMEM((m + 1, N, W), _F32)],
        compiler_params=pltpu.CompilerParams(
            dimension_semantics=("arbitrary", "arbitrary"),
            vmem_limit_bytes=100 * 1024 * 1024),
    )(*operands)


def _logits_body(h_ref, w_ref, b_ref, o_ref):
    o_ref[...] = jax.nn.sigmoid(
        jnp.dot(_b(h_ref[...]), _b(w_ref[...]), preferred_element_type=_F32) + b_ref[...])


def _logits(h, W, b):
    return pl.pallas_call(
        _logits_body,
        out_shape=jax.ShapeDtypeStruct((h.shape[0], W.shape[1]), _F32),
    )(h, W, b.reshape(1, -1))


def kernel(x_0, x_1, x_2, laplacian_0, laplacian_down_1, laplacian_up_1,
           laplacian_2, incidence_1, incidence_2, in_W0, in_b0, in_W1, in_b1,
           in_W2, in_b2, w0_l0, w1_l0, w2_l0, w0_l1, w1_l1, w2_l1,
           out_W, out_b):
    h0, h1, h2 = _embed(x_0, x_1, x_2, in_W0, in_b0, in_W1, in_b1, in_W2, in_b2)
    idx_d = jnp.array([0, 1, 2, 5, 6, 7, 10, 11, 12], jnp.int32)
    idx_u = jnp.array([3, 4, 8, 9, 13, 14], jnp.int32)
    for (w0, w1, w2) in ((w0_l0, w1_l0, w2_l0), (w0_l1, w1_l1, w2_l1)):
        t01, t10 = _incidence(incidence_1, h1, h0)
        t12, t21 = _incidence(incidence_2, h2, h1)
        w0t = jnp.transpose(w0, (2, 0, 1))
        w1t = jnp.transpose(w1, (2, 0, 1))
        w2t = jnp.transpose(w2, (2, 0, 1))
        y0 = _cheby_fused(laplacian_0, [h0, t01], w0t, include_id=True)
        y1d = _cheby_fused(laplacian_down_1, [h1, t10, t12], w1t[idx_d],
                           include_id=True)
        y1 = _cheby_fused(laplacian_up_1, [h1, t10, t12], w1t[idx_u],
                          include_id=False, init=y1d)
        y2 = _cheby_fused(laplacian_2, [h2, t21], w2t, include_id=True)
        h0, h1, h2 = y0, y1, y2
    return _logits(h0, out_W, out_b)


# speedup vs baseline: 1.1253x; 1.0495x over previous
import jax
import jax.numpy as jnp
from jax.experimental import pallas as pl
from jax.experimental.pallas import tpu as pltpu

_F32 = jnp.float32
_BF16 = jnp.bfloat16


def _b(v):
    return v.astype(_BF16)


def _embed_body(x0, x1, x2, w0, b0, w1, b1, w2, b2, h0, h1, h2):
    h0[...] = jnp.dot(_b(x0[...]), _b(w0[...]), preferred_element_type=_F32) + b0[...]
    h1[...] = jnp.dot(_b(x1[...]), _b(w1[...]), preferred_element_type=_F32) + b1[...]
    h2[...] = jnp.dot(_b(x2[...]), _b(w2[...]), preferred_element_type=_F32) + b2[...]


def _embed(x0, x1, x2, W0, b0, W1, b1, W2, b2):
    C = W0.shape[1]
    outs = [jax.ShapeDtypeStruct((x.shape[0], C), _F32) for x in (x0, x1, x2)]
    return pl.pallas_call(_embed_body, out_shape=outs)(
        x0, x1, x2, W0, b0.reshape(1, -1), W1, b1.reshape(1, -1), W2, b2.reshape(1, -1)
    )


def _inc_body(B_ref, xs_ref, xd_ref, f_ref, bwd_ref):
    i = pl.program_id(0)
    blk = _b(B_ref[...])
    f_ref[...] = jnp.dot(blk, _b(xs_ref[...]), preferred_element_type=_F32)

    @pl.when(i == 0)
    def _():
        bwd_ref[...] = jnp.zeros_like(bwd_ref)

    bwd_ref[...] += jax.lax.dot_general(
        blk, _b(xd_ref[...]), dimension_numbers=(((0,), (0,)), ((), ())),
        preferred_element_type=_F32)


def _incidence(B, xs, xd, R=256):
    Nr, Nc = B.shape
    C = xs.shape[1]
    return pl.pallas_call(
        _inc_body,
        grid=(Nr // R,),
        in_specs=[
            pl.BlockSpec((R, Nc), lambda i: (i, 0)),
            pl.BlockSpec((Nc, C), lambda i: (0, 0)),
            pl.BlockSpec((R, C), lambda i: (i, 0)),
        ],
        out_specs=[
            pl.BlockSpec((R, C), lambda i: (i, 0)),
            pl.BlockSpec((Nc, C), lambda i: (0, 0)),
        ],
        out_shape=[
            jax.ShapeDtypeStruct((Nr, C), _F32),
            jax.ShapeDtypeStruct((Nc, C), _F32),
        ],
        compiler_params=pltpu.CompilerParams(dimension_semantics=("arbitrary",)),
    )(B, xs, xd)


def _cheby_fused(Ls, srcs, wt, R=256):
    n_ops, n_src = len(Ls), len(srcs)
    N = Ls[0].shape[0]
    C = srcs[0].shape[1]
    W = C * n_src
    K, _, C_OUT = wt.shape
    m = (K // n_src - 1) // n_ops
    nR = N // R

    def body(*refs):
        L_refs = refs[:n_ops]
        src_refs = refs[n_ops:n_ops + n_src]
        wt_ref = refs[n_ops + n_src]
        y_ref = refs[n_ops + n_src + 1]
        Lbs = refs[n_ops + n_src + 2:n_ops + n_src + 2 + n_ops]
        chain = refs[2 * n_ops + n_src + 2]
        p = pl.program_id(0)
        i = pl.program_id(1)

        @pl.when((p == 0) & (i == 0))
        def _():
            for s in range(n_src):
                chain[0, :, s * C:(s + 1) * C] = _b(src_refs[s][...])

        rows = pl.ds(i * R, R)

        @pl.when(p == 0)
        def _():
            for o in range(n_ops):
                blk = _b(L_refs[o][...])
                Lbs[o][rows, :] = blk
                chain[1 + o * m, rows, :] = _b(
                    jnp.dot(blk, chain[0], preferred_element_type=_F32))

        @pl.when(p > 0)
        def _():
            for o in range(n_ops):
                chain[1 + o * m + p, rows, :] = _b(
                    jnp.dot(Lbs[o][rows, :], chain[o * m + p],
                            preferred_element_type=_F32))

        @pl.when(p == m - 1)
        def _():
            acc = jnp.zeros((R, C_OUT), _F32)
            k = 0
            for s in range(n_src):
                cs = slice(s * C, (s + 1) * C)
                acc += jnp.dot(chain[0, rows, cs], wt_ref[k],
                               preferred_element_type=_F32)
                k += 1
                for o in range(n_ops):
                    for j in range(1, m + 1):
                        acc += jnp.dot(chain[o * m + j, rows, cs], wt_ref[k],
                                       preferred_element_type=_F32)
                        k += 1
            y_ref[rows, :] = acc

    in_specs = (
        [pl.BlockSpec((R, N), lambda p, i: (jnp.where(p == 0, i, 0), 0))
         for _ in Ls]
        + [pl.BlockSpec((N, C), lambda p, i: (0, 0)) for _ in srcs]
        + [pl.BlockSpec(wt.shape, lambda p, i: (0, 0, 0))]
    )
    return pl.pallas_call(
        body,
        grid=(m, nR),
        in_specs=in_specs,
        out_specs=pl.BlockSpec((N, C_OUT), lambda p, i: (0, 0)),
        out_shape=jax.ShapeDtypeStruct((N, C_OUT), _F32),
        scratch_shapes=([pltpu.VMEM((N, N), _BF16) for _ in Ls]
                        + [pltpu.VMEM((1 + n_ops * m, N, W), _BF16)]),
        compiler_params=pltpu.CompilerParams(
            dimension_semantics=("arbitrary", "arbitrary"),
            vmem_limit_bytes=100 * 1024 * 1024),
    )(*Ls, *srcs, wt)


def _logits_body(h_ref, w_ref, b_ref, o_ref):
    o_ref[...] = jax.nn.sigmoid(
        jnp.dot(_b(h_ref[...]), _b(w_ref[...]), preferred_element_type=_F32) + b_ref[...])


def _logits(h, W, b):
    return pl.pallas_call(
        _logits_body,
        out_shape=jax.ShapeDtypeStruct((h.shape[0], W.shape[1]), _F32),
    )(h, W, b.reshape(1, -1))


def kernel(x_0, x_1, x_2, laplacian_0, laplacian_down_1, laplacian_up_1,
           laplacian_2, incidence_1, incidence_2, in_W0, in_b0, in_W1, in_b1,
           in_W2, in_b2, w0_l0, w1_l0, w2_l0, w0_l1, w1_l1, w2_l1,
           out_W, out_b):
    h0, h1, h2 = _embed(x_0, x_1, x_2, in_W0, in_b0, in_W1, in_b1, in_W2, in_b2)
    for (w0, w1, w2) in ((w0_l0, w1_l0, w2_l0), (w0_l1, w1_l1, w2_l1)):
        t01, t10 = _incidence(incidence_1, h1, h0, R=512)
        t12, t21 = _incidence(incidence_2, h2, h1, R=512)
        w0t = _b(jnp.transpose(w0, (2, 0, 1)))
        w1t = _b(jnp.transpose(w1, (2, 0, 1)))
        w2t = _b(jnp.transpose(w2, (2, 0, 1)))
        y0 = _cheby_fused([laplacian_0], [h0, t01], w0t, R=512)
        y1 = _cheby_fused([laplacian_down_1, laplacian_up_1], [h1, t10, t12],
                          w1t, R=256)
        y2 = _cheby_fused([laplacian_2], [h2, t21], w2t, R=512)
        h0, h1, h2 = y0, y1, y2
    return _logits(h0, out_W, out_b)


# speedup vs baseline: 1.1510x; 1.0229x over previous
import jax
import jax.numpy as jnp
from jax.experimental import pallas as pl
from jax.experimental.pallas import tpu as pltpu

_F32 = jnp.float32
_BF16 = jnp.bfloat16


def _b(v):
    return v.astype(_BF16)


def _embed_body(x0, x1, x2, w0, b0, w1, b1, w2, b2, h0, h1, h2):
    h0[...] = jnp.dot(_b(x0[...]), _b(w0[...]), preferred_element_type=_F32) + b0[...]
    h1[...] = jnp.dot(_b(x1[...]), _b(w1[...]), preferred_element_type=_F32) + b1[...]
    h2[...] = jnp.dot(_b(x2[...]), _b(w2[...]), preferred_element_type=_F32) + b2[...]


def _embed(x0, x1, x2, W0, b0, W1, b1, W2, b2):
    C = W0.shape[1]
    outs = [jax.ShapeDtypeStruct((x.shape[0], C), _F32) for x in (x0, x1, x2)]
    return pl.pallas_call(_embed_body, out_shape=outs)(
        x0, x1, x2, W0, b0.reshape(1, -1), W1, b1.reshape(1, -1), W2, b2.reshape(1, -1)
    )


def _inc_body(B_ref, xs_ref, xd_ref, f_ref, bwd_ref):
    i = pl.program_id(0)
    blk = _b(B_ref[...])
    f_ref[...] = jnp.dot(blk, _b(xs_ref[...]), preferred_element_type=_F32)

    @pl.when(i == 0)
    def _():
        bwd_ref[...] = jnp.zeros_like(bwd_ref)

    bwd_ref[...] += jax.lax.dot_general(
        blk, _b(xd_ref[...]), dimension_numbers=(((0,), (0,)), ((), ())),
        preferred_element_type=_F32)


def _incidence(B, xs, xd, R=256):
    Nr, Nc = B.shape
    C = xs.shape[1]
    return pl.pallas_call(
        _inc_body,
        grid=(Nr // R,),
        in_specs=[
            pl.BlockSpec((R, Nc), lambda i: (i, 0)),
            pl.BlockSpec((Nc, C), lambda i: (0, 0)),
            pl.BlockSpec((R, C), lambda i: (i, 0)),
        ],
        out_specs=[
            pl.BlockSpec((R, C), lambda i: (i, 0)),
            pl.BlockSpec((Nc, C), lambda i: (0, 0)),
        ],
        out_shape=[
            jax.ShapeDtypeStruct((Nr, C), _F32),
            jax.ShapeDtypeStruct((Nc, C), _F32),
        ],
        compiler_params=pltpu.CompilerParams(dimension_semantics=("arbitrary",)),
    )(B, xs, xd)


def _cheby_fused(Ls, srcs, wt, R=256):
    n_ops, n_src = len(Ls), len(srcs)
    N = Ls[0].shape[0]
    C = srcs[0].shape[1]
    W = C * n_src
    n_slots = wt.shape[0] // 128
    m = (n_slots - 1) // n_ops
    C_OUT = wt.shape[1]
    nR = N // R

    def body(*refs):
        L_refs = refs[:n_ops]
        src_refs = refs[n_ops:n_ops + n_src]
        wt_ref = refs[n_ops + n_src]
        y_ref = refs[n_ops + n_src + 1]
        Lbs = refs[n_ops + n_src + 2:n_ops + n_src + 2 + n_ops]
        chain = refs[2 * n_ops + n_src + 2]
        p = pl.program_id(0)
        i = pl.program_id(1)

        @pl.when((p == 0) & (i == 0))
        def _():
            chain[...] = jnp.zeros_like(chain)
            for s in range(n_src):
                chain[:, s * C:(s + 1) * C] = _b(src_refs[s][...])

        rows = pl.ds(i * R, R)

        @pl.when(p == 0)
        def _():
            for o in range(n_ops):
                blk = _b(L_refs[o][...])
                Lbs[o][rows, :] = blk
                sl = 1 + o * m
                chain[rows, sl * 128:sl * 128 + W] = _b(
                    jnp.dot(blk, chain[:, 0:W], preferred_element_type=_F32))

        @pl.when(p > 0)
        def _():
            for o in range(n_ops):
                src = chain[:, pl.ds((o * m + p) * 128, W)]
                chain[rows, pl.ds((1 + o * m + p) * 128, W)] = _b(
                    jnp.dot(Lbs[o][rows, :], src, preferred_element_type=_F32))

        @pl.when(p == m - 1)
        def _():
            y_ref[rows, :] = jnp.dot(chain[rows, :], wt_ref[...],
                                     preferred_element_type=_F32)

    in_specs = (
        [pl.BlockSpec((R, N), lambda p, i: (jnp.where(p == 0, i, 0), 0))
         for _ in Ls]
        + [pl.BlockSpec((N, C), lambda p, i: (0, 0)) for _ in srcs]
        + [pl.BlockSpec(wt.shape, lambda p, i: (0, 0))]
    )
    return pl.pallas_call(
        body,
        grid=(m, nR),
        in_specs=in_specs,
        out_specs=pl.BlockSpec((N, C_OUT), lambda p, i: (0, 0)),
        out_shape=jax.ShapeDtypeStruct((N, C_OUT), _F32),
        scratch_shapes=([pltpu.VMEM((N, N), _BF16) for _ in Ls]
                        + [pltpu.VMEM((N, n_slots * 128), _BF16)]),
        compiler_params=pltpu.CompilerParams(
            dimension_semantics=("arbitrary", "arbitrary"),
            vmem_limit_bytes=100 * 1024 * 1024),
    )(*Ls, *srcs, wt)


def _stack_weights(wt, n_src, C):
    K, _, C_OUT = wt.shape
    n_slots = K // n_src
    w = wt.reshape(n_src, n_slots, C, C_OUT).transpose(1, 0, 2, 3)
    w = w.reshape(n_slots, n_src * C, C_OUT)
    w = jnp.pad(w, ((0, 0), (0, 128 - n_src * C), (0, 0)))
    return _b(w.reshape(n_slots * 128, C_OUT))


def _logits_body(h_ref, w_ref, b_ref, o_ref):
    o_ref[...] = jax.nn.sigmoid(
        jnp.dot(_b(h_ref[...]), _b(w_ref[...]), preferred_element_type=_F32) + b_ref[...])


def _logits(h, W, b):
    return pl.pallas_call(
        _logits_body,
        out_shape=jax.ShapeDtypeStruct((h.shape[0], W.shape[1]), _F32),
    )(h, W, b.reshape(1, -1))


def kernel(x_0, x_1, x_2, laplacian_0, laplacian_down_1, laplacian_up_1,
           laplacian_2, incidence_1, incidence_2, in_W0, in_b0, in_W1, in_b1,
           in_W2, in_b2, w0_l0, w1_l0, w2_l0, w0_l1, w1_l1, w2_l1,
           out_W, out_b):
    h0, h1, h2 = _embed(x_0, x_1, x_2, in_W0, in_b0, in_W1, in_b1, in_W2, in_b2)
    for (w0, w1, w2) in ((w0_l0, w1_l0, w2_l0), (w0_l1, w1_l1, w2_l1)):
        t01, t10 = _incidence(incidence_1, h1, h0, R=512)
        t12, t21 = _incidence(incidence_2, h2, h1, R=512)
        w0t = _stack_weights(jnp.transpose(w0, (2, 0, 1)), 2, 32)
        w1t = _stack_weights(jnp.transpose(w1, (2, 0, 1)), 3, 32)
        w2t = _stack_weights(jnp.transpose(w2, (2, 0, 1)), 2, 32)
        y0 = _cheby_fused([laplacian_0], [h0, t01], w0t, R=512)
        y1 = _cheby_fused([laplacian_down_1, laplacian_up_1], [h1, t10, t12],
                          w1t, R=256)
        y2 = _cheby_fused([laplacian_2], [h2, t21], w2t, R=512)
        h0, h1, h2 = y0, y1, y2
    return _logits(h0, out_W, out_b)


# speedup vs baseline: 1.3410x; 1.1650x over previous
import jax
import jax.numpy as jnp
from jax.experimental import pallas as pl
from jax.experimental.pallas import tpu as pltpu

_F32 = jnp.float32
_BF16 = jnp.bfloat16


def _b(v):
    return v.astype(_BF16)


def _embed_body(x0, x1, x2, w0, b0, w1, b1, w2, b2, h0, h1, h2):
    h0[...] = jnp.dot(_b(x0[...]), _b(w0[...]), preferred_element_type=_F32) + b0[...]
    h1[...] = jnp.dot(_b(x1[...]), _b(w1[...]), preferred_element_type=_F32) + b1[...]
    h2[...] = jnp.dot(_b(x2[...]), _b(w2[...]), preferred_element_type=_F32) + b2[...]


def _embed(x0, x1, x2, W0, b0, W1, b1, W2, b2):
    C = W0.shape[1]
    outs = [jax.ShapeDtypeStruct((x.shape[0], C), _F32) for x in (x0, x1, x2)]
    return pl.pallas_call(_embed_body, out_shape=outs)(
        x0, x1, x2, W0, b0.reshape(1, -1), W1, b1.reshape(1, -1), W2, b2.reshape(1, -1)
    )


def _inc_body(B_ref, xs_ref, xd_ref, f_ref, bwd_ref):
    i = pl.program_id(0)
    blk = _b(B_ref[...])
    f_ref[...] = jnp.dot(blk, _b(xs_ref[...]), preferred_element_type=_F32)

    @pl.when(i == 0)
    def _():
        bwd_ref[...] = jnp.zeros_like(bwd_ref)

    bwd_ref[...] += jax.lax.dot_general(
        blk, _b(xd_ref[...]), dimension_numbers=(((0,), (0,)), ((), ())),
        preferred_element_type=_F32)


def _inc_fwd_body(B_ref, xs_ref, f_ref):
    f_ref[...] = jnp.dot(_b(B_ref[...]), _b(xs_ref[...]),
                         preferred_element_type=_F32)


def _incidence_fwd(B, xs, R=512):
    Nr, Nc = B.shape
    C = xs.shape[1]
    return pl.pallas_call(
        _inc_fwd_body,
        grid=(Nr // R,),
        in_specs=[
            pl.BlockSpec((R, Nc), lambda i: (i, 0)),
            pl.BlockSpec((Nc, C), lambda i: (0, 0)),
        ],
        out_specs=pl.BlockSpec((R, C), lambda i: (i, 0)),
        out_shape=jax.ShapeDtypeStruct((Nr, C), _F32),
        compiler_params=pltpu.CompilerParams(dimension_semantics=("arbitrary",)),
    )(B, xs)


def _incidence(B, xs, xd, R=256):
    Nr, Nc = B.shape
    C = xs.shape[1]
    return pl.pallas_call(
        _inc_body,
        grid=(Nr // R,),
        in_specs=[
            pl.BlockSpec((R, Nc), lambda i: (i, 0)),
            pl.BlockSpec((Nc, C), lambda i: (0, 0)),
            pl.BlockSpec((R, C), lambda i: (i, 0)),
        ],
        out_specs=[
            pl.BlockSpec((R, C), lambda i: (i, 0)),
            pl.BlockSpec((Nc, C), lambda i: (0, 0)),
        ],
        out_shape=[
            jax.ShapeDtypeStruct((Nr, C), _F32),
            jax.ShapeDtypeStruct((Nc, C), _F32),
        ],
        compiler_params=pltpu.CompilerParams(dimension_semantics=("arbitrary",)),
    )(B, xs, xd)


def _cheby_fused(Ls, srcs, wt, R=256):
    n_ops, n_src = len(Ls), len(srcs)
    N = Ls[0].shape[0]
    C = srcs[0].shape[1]
    W = C * n_src
    n_slots = wt.shape[0] // 128
    m = (n_slots - 1) // n_ops
    C_OUT = wt.shape[1]
    nR = N // R

    def body(*refs):
        L_refs = refs[:n_ops]
        src_refs = refs[n_ops:n_ops + n_src]
        wt_ref = refs[n_ops + n_src]
        y_ref = refs[n_ops + n_src + 1]
        Lbs = refs[n_ops + n_src + 2:n_ops + n_src + 2 + n_ops]
        chain = refs[2 * n_ops + n_src + 2]
        p = pl.program_id(0)
        i = pl.program_id(1)

        @pl.when((p == 0) & (i == 0))
        def _():
            chain[...] = jnp.zeros_like(chain)
            for s in range(n_src):
                chain[:, s * C:(s + 1) * C] = _b(src_refs[s][...])

        rows = pl.ds(i * R, R)

        @pl.when(p == 0)
        def _():
            for o in range(n_ops):
                blk = _b(L_refs[o][...])
                Lbs[o][rows, :] = blk
                sl = 1 + o * m
                chain[rows, sl * 128:sl * 128 + W] = _b(
                    jnp.dot(blk, chain[:, 0:W], preferred_element_type=_F32))

        @pl.when(p > 0)
        def _():
            for o in range(n_ops):
                src = chain[:, pl.ds((o * m + p) * 128, W)]
                chain[rows, pl.ds((1 + o * m + p) * 128, W)] = _b(
                    jnp.dot(Lbs[o][rows, :], src, preferred_element_type=_F32))

        @pl.when(p == m - 1)
        def _():
            y_ref[rows, :] = jnp.dot(chain[rows, :], wt_ref[...],
                                     preferred_element_type=_F32)

    in_specs = (
        [pl.BlockSpec((R, N), lambda p, i: (jnp.where(p == 0, i, 0), 0))
         for _ in Ls]
        + [pl.BlockSpec((N, C), lambda p, i: (0, 0)) for _ in srcs]
        + [pl.BlockSpec(wt.shape, lambda p, i: (0, 0))]
    )
    return pl.pallas_call(
        body,
        grid=(m, nR),
        in_specs=in_specs,
        out_specs=pl.BlockSpec((N, C_OUT), lambda p, i: (0, 0)),
        out_shape=jax.ShapeDtypeStruct((N, C_OUT), _F32),
        scratch_shapes=([pltpu.VMEM((N, N), _BF16) for _ in Ls]
                        + [pltpu.VMEM((N, n_slots * 128), _BF16)]),
        compiler_params=pltpu.CompilerParams(
            dimension_semantics=("arbitrary", "arbitrary"),
            vmem_limit_bytes=100 * 1024 * 1024),
    )(*Ls, *srcs, wt)


def _stack_weights(wt, n_src, C):
    K, _, C_OUT = wt.shape
    n_slots = K // n_src
    w = wt.reshape(n_src, n_slots, C, C_OUT).transpose(1, 0, 2, 3)
    w = w.reshape(n_slots, n_src * C, C_OUT)
    w = jnp.pad(w, ((0, 0), (0, 128 - n_src * C), (0, 0)))
    return _b(w.reshape(n_slots * 128, C_OUT))


def _logits_body(h_ref, w_ref, b_ref, o_ref):
    o_ref[...] = jax.nn.sigmoid(
        jnp.dot(_b(h_ref[...]), _b(w_ref[...]), preferred_element_type=_F32) + b_ref[...])


def _logits(h, W, b):
    return pl.pallas_call(
        _logits_body,
        out_shape=jax.ShapeDtypeStruct((h.shape[0], W.shape[1]), _F32),
    )(h, W, b.reshape(1, -1))


def kernel(x_0, x_1, x_2, laplacian_0, laplacian_down_1, laplacian_up_1,
           laplacian_2, incidence_1, incidence_2, in_W0, in_b0, in_W1, in_b1,
           in_W2, in_b2, w0_l0, w1_l0, w2_l0, w0_l1, w1_l1, w2_l1,
           out_W, out_b):
    h0, h1, h2 = _embed(x_0, x_1, x_2, in_W0, in_b0, in_W1, in_b1, in_W2, in_b2)

    t01, t10 = _incidence(incidence_1, h1, h0, R=512)
    t12 = _incidence_fwd(incidence_2, h2, R=512)
    y0 = _cheby_fused([laplacian_0], [h0, t01],
                      _stack_weights(jnp.transpose(w0_l0, (2, 0, 1)), 2, 32),
                      R=512)
    y1 = _cheby_fused([laplacian_down_1, laplacian_up_1], [h1, t10, t12],
                      _stack_weights(jnp.transpose(w1_l0, (2, 0, 1)), 3, 32),
                      R=256)
    h0, h1 = y0, y1

    t01 = _incidence_fwd(incidence_1, h1, R=512)
    h0 = _cheby_fused([laplacian_0], [h0, t01],
                      _stack_weights(jnp.transpose(w0_l1, (2, 0, 1)), 2, 32),
                      R=512)
    return _logits(h0, out_W, out_b)
